# Initial kernel scaffold; baseline (speedup 1.0000x reference)
import jax
import jax.numpy as jnp
from jax.experimental import pallas as pl


def kernel(batch_size, head_count):
    raise NotImplementedError("write your pallas kernel here")


# speedup vs baseline: 6.6119x; 6.6119x over previous
import jax
import jax.numpy as jnp
from jax import lax
from jax.experimental import pallas as pl
from jax.experimental.pallas import tpu as pltpu
from jax.experimental.pallas import tpu_sc as plsc

N = 32
BH = 128
TBL = N * N
NW = 32
LANES = 16


def _body(kv_num, kv_idx, fkv_num, fkv_idx, q_num, q_idx, fq_num, fq_idx,
          kv_tbl, fkv_tbl, fq_tbl, one_row, inc_row, dec_row):
    wid = lax.axis_index("s") * 2 + lax.axis_index("c")
    iota = lax.iota(jnp.int32, LANES)

    for i in range(N):
        for h in range(N // LANES):
            c = iota + h * LANES
            sl = pl.ds(i * N + h * LANES, LANES)
            kv_tbl[sl] = jnp.where(c == 0, i, jnp.where(c <= i, c - 1, c))
            fkv_tbl[sl] = c
            fq_tbl[sl] = jnp.where(c < N - 1 - i, i + 1 + c, c - (N - 1 - i))

    for j in range(BH // LANES):
        c = iota + j * LANES
        r = lax.rem(c, N)
        sl = pl.ds(j * LANES, LANES)
        one_row[sl] = r * 0 + 1
        inc_row[sl] = r
        dec_row[sl] = (N - 1) - r

    base = wid * (BH // NW)
    for k in range(BH // NW):
        off = (base + k) * TBL
        dst = pl.ds(off, TBL)
        pltpu.sync_copy(kv_tbl, kv_idx.at[dst])
        pltpu.sync_copy(fkv_tbl, fkv_idx.at[dst])
        pltpu.sync_copy(kv_tbl, q_idx.at[dst])
        pltpu.sync_copy(fq_tbl, fq_idx.at[dst])

    nb = pl.ds(wid * (BH // NW) * N, (BH // NW) * N)
    pltpu.sync_copy(one_row, kv_num.at[nb])
    pltpu.sync_copy(inc_row, fkv_num.at[nb])
    pltpu.sync_copy(one_row, q_num.at[nb])
    pltpu.sync_copy(dec_row, fq_num.at[nb])


_NUM_T = jax.ShapeDtypeStruct((BH * N,), jnp.int32)
_IDX_T = jax.ShapeDtypeStruct((BH * TBL,), jnp.int32)

_sc_prog = pl.kernel(
    _body,
    out_type=(_NUM_T, _IDX_T, _NUM_T, _IDX_T, _NUM_T, _IDX_T, _NUM_T, _IDX_T),
    mesh=plsc.VectorSubcoreMesh(core_axis_name="c", subcore_axis_name="s"),
    scratch_types=[
        pltpu.VMEM((TBL,), jnp.int32),
        pltpu.VMEM((TBL,), jnp.int32),
        pltpu.VMEM((TBL,), jnp.int32),
        pltpu.VMEM((BH // NW * N,), jnp.int32),
        pltpu.VMEM((BH // NW * N,), jnp.int32),
        pltpu.VMEM((BH // NW * N,), jnp.int32),
    ],
)


def kernel(batch_size, head_count):
    B = batch_size.shape[0]
    H = head_count.shape[0]
    kvn, kvi, fkvn, fkvi, qn, qi, fqn, fqi = _sc_prog()
    s3 = (B, H, N)
    s4 = (B, H, N, N)
    return (
        kvn.reshape(s3),
        kvi.reshape(s4),
        fkvn.reshape(s3),
        fkvi.reshape(s4),
        qn.reshape(s3),
        qi.reshape(s4),
        fqn.reshape(s3),
        fqi.reshape(s4),
    )


# speedup vs baseline: 6.7589x; 1.0222x over previous
import jax
import jax.numpy as jnp
from jax import lax
from jax.experimental import pallas as pl
from jax.experimental.pallas import tpu as pltpu
from jax.experimental.pallas import tpu_sc as plsc

N = 32
BH = 128
TBL = N * N
NW = 32
LANES = 16


def _body(kv_num, kv_idx, fkv_num, fkv_idx, q_num, q_idx, fq_num, fq_idx,
          kv_tbl, fkv_tbl, fq_tbl, one_row, inc_row, dec_row, sem):
    wid = lax.axis_index("s") * 2 + lax.axis_index("c")
    iota = lax.iota(jnp.int32, LANES)

    for i in range(N):
        for h in range(N // LANES):
            c = iota + h * LANES
            sl = pl.ds(i * N + h * LANES, LANES)
            kv_tbl[sl] = jnp.where(c == 0, i, jnp.where(c <= i, c - 1, c))
            fkv_tbl[sl] = c
            fq_tbl[sl] = jnp.where(c < N - 1 - i, i + 1 + c, c - (N - 1 - i))

    for j in range(BH // LANES):
        c = iota + j * LANES
        r = lax.rem(c, N)
        sl = pl.ds(j * LANES, LANES)
        one_row[sl] = r * 0 + 1
        inc_row[sl] = r
        dec_row[sl] = (N - 1) - r

    base = wid * (BH // NW)
    handles = []
    for k in range(BH // NW):
        off = (base + k) * TBL
        dst = pl.ds(off, TBL)
        handles.append(pltpu.async_copy(kv_tbl, kv_idx.at[dst], sem))
        handles.append(pltpu.async_copy(fkv_tbl, fkv_idx.at[dst], sem))
        handles.append(pltpu.async_copy(kv_tbl, q_idx.at[dst], sem))
        handles.append(pltpu.async_copy(fq_tbl, fq_idx.at[dst], sem))

    nb = pl.ds(wid * (BH // NW) * N, (BH // NW) * N)
    handles.append(pltpu.async_copy(one_row, kv_num.at[nb], sem))
    handles.append(pltpu.async_copy(inc_row, fkv_num.at[nb], sem))
    handles.append(pltpu.async_copy(one_row, q_num.at[nb], sem))
    handles.append(pltpu.async_copy(dec_row, fq_num.at[nb], sem))
    for h in handles:
        h.wait()


_NUM_T = jax.ShapeDtypeStruct((BH * N,), jnp.int32)
_IDX_T = jax.ShapeDtypeStruct((BH * TBL,), jnp.int32)

_sc_prog = pl.kernel(
    _body,
    out_type=(_NUM_T, _IDX_T, _NUM_T, _IDX_T, _NUM_T, _IDX_T, _NUM_T, _IDX_T),
    mesh=plsc.VectorSubcoreMesh(core_axis_name="c", subcore_axis_name="s"),
    scratch_types=[
        pltpu.VMEM((TBL,), jnp.int32),
        pltpu.VMEM((TBL,), jnp.int32),
        pltpu.VMEM((TBL,), jnp.int32),
        pltpu.VMEM((BH // NW * N,), jnp.int32),
        pltpu.VMEM((BH // NW * N,), jnp.int32),
        pltpu.VMEM((BH // NW * N,), jnp.int32),
        pltpu.SemaphoreType.DMA,
    ],
)


def kernel(batch_size, head_count):
    B = batch_size.shape[0]
    H = head_count.shape[0]
    kvn, kvi, fkvn, fkvi, qn, qi, fqn, fqi = _sc_prog()
    s3 = (B, H, N)
    s4 = (B, H, N, N)
    return (
        kvn.reshape(s3),
        kvi.reshape(s4),
        fkvn.reshape(s3),
        fkvi.reshape(s4),
        qn.reshape(s3),
        qi.reshape(s4),
        fqn.reshape(s3),
        fqi.reshape(s4),
    )


# speedup vs baseline: 6.8818x; 1.0182x over previous
import jax
import jax.numpy as jnp
from jax import lax
from jax.experimental import pallas as pl
from jax.experimental.pallas import tpu as pltpu
from jax.experimental.pallas import tpu_sc as plsc

N = 32
BH = 128
TBL = N * N
NW = 32
LANES = 16


def _body(kv_num, kv_idx, fkv_num, fkv_idx, q_num, q_idx, fq_num, fq_idx,
          kv_tbl, fkv_tbl, fq_tbl, one_row, inc_row, dec_row, sem):
    wid = lax.axis_index("s") * 2 + lax.axis_index("c")
    iota = lax.iota(jnp.int32, LANES)
    base = wid * (BH // NW)
    slices = [pl.ds((base + k) * TBL, TBL) for k in range(BH // NW)]
    handles = []

    for i in range(N):
        for h in range(N // LANES):
            c = iota + h * LANES
            sl = pl.ds(i * N + h * LANES, LANES)
            kv_tbl[sl] = jnp.where(c == 0, i, jnp.where(c <= i, c - 1, c))
    for dst in slices:
        handles.append(pltpu.async_copy(kv_tbl, kv_idx.at[dst], sem))
        handles.append(pltpu.async_copy(kv_tbl, q_idx.at[dst], sem))

    for i in range(N):
        for h in range(N // LANES):
            c = iota + h * LANES
            sl = pl.ds(i * N + h * LANES, LANES)
            fq_tbl[sl] = jnp.where(c < N - 1 - i, i + 1 + c, c - (N - 1 - i))
    for dst in slices:
        handles.append(pltpu.async_copy(fq_tbl, fq_idx.at[dst], sem))

    for h in range(N // LANES):
        c = iota + h * LANES
        for i in range(N):
            fkv_tbl[pl.ds(i * N + h * LANES, LANES)] = c
    for dst in slices:
        handles.append(pltpu.async_copy(fkv_tbl, fkv_idx.at[dst], sem))

    for j in range(BH // LANES):
        c = iota + j * LANES
        r = lax.rem(c, N)
        sl = pl.ds(j * LANES, LANES)
        one_row[sl] = r * 0 + 1
        inc_row[sl] = r
        dec_row[sl] = (N - 1) - r
    nb = pl.ds(wid * (BH // NW) * N, (BH // NW) * N)
    handles.append(pltpu.async_copy(one_row, kv_num.at[nb], sem))
    handles.append(pltpu.async_copy(inc_row, fkv_num.at[nb], sem))
    handles.append(pltpu.async_copy(one_row, q_num.at[nb], sem))
    handles.append(pltpu.async_copy(dec_row, fq_num.at[nb], sem))
    for hd in handles:
        hd.wait()


_NUM_T = jax.ShapeDtypeStruct((BH * N,), jnp.int32)
_IDX_T = jax.ShapeDtypeStruct((BH * TBL,), jnp.int32)

_sc_prog = pl.kernel(
    _body,
    out_type=(_NUM_T, _IDX_T, _NUM_T, _IDX_T, _NUM_T, _IDX_T, _NUM_T, _IDX_T),
    mesh=plsc.VectorSubcoreMesh(core_axis_name="c", subcore_axis_name="s"),
    scratch_types=[
        pltpu.VMEM((TBL,), jnp.int32),
        pltpu.VMEM((TBL,), jnp.int32),
        pltpu.VMEM((TBL,), jnp.int32),
        pltpu.VMEM((BH // NW * N,), jnp.int32),
        pltpu.VMEM((BH // NW * N,), jnp.int32),
        pltpu.VMEM((BH // NW * N,), jnp.int32),
        pltpu.SemaphoreType.DMA,
    ],
)


def kernel(batch_size, head_count):
    B = batch_size.shape[0]
    H = head_count.shape[0]
    kvn, kvi, fkvn, fkvi, qn, qi, fqn, fqi = _sc_prog()
    s3 = (B, H, N)
    s4 = (B, H, N, N)
    return (
        kvn.reshape(s3),
        kvi.reshape(s4),
        fkvn.reshape(s3),
        fkvi.reshape(s4),
        qn.reshape(s3),
        qi.reshape(s4),
        fqn.reshape(s3),
        fqi.reshape(s4),
    )
